# baseline (speedup 1.0000x reference)
import functools

import jax
import jax.numpy as jnp
from jax import lax
from jax.experimental import pallas as pl
from jax.experimental.pallas import tpu as pltpu
from jax.experimental.pallas import tpu_sc as plsc

_N = 10000
_E = 320000
_B = 128
_NBAT = _E // _B
_NT = 32
_BASE_BAT = _NBAT // _NT
_EXTRA = _NBAT - _BASE_BAT * _NT
_NPAD = 10240
_ZR = _NPAD // 16
_DW = 128
_RB = 1000
_G = _N // _RB


def _tile_batches(wid):
  nbat = _BASE_BAT + (wid < _EXTRA).astype(jnp.int32)
  base = wid * _BASE_BAT + jnp.minimum(wid, _EXTRA)
  return nbat, base


def _make_seg_sum(D):
  mesh = plsc.VectorSubcoreMesh(core_axis_name="c", subcore_axis_name="s")
  out_type = [jax.ShapeDtypeStruct((_NPAD, D), jnp.float32),
              jax.ShapeDtypeStruct((_NPAD, D), jnp.float32)]
  scratch = [
      pltpu.VMEM((_B,), jnp.int32),
      pltpu.VMEM((_B,), jnp.int32),
      pltpu.VMEM((_B,), jnp.int32),
      pltpu.VMEM((_B,), jnp.int32),
      pltpu.VMEM((_B, D), jnp.float32),
      pltpu.VMEM((_B, D), jnp.float32),
      pltpu.VMEM_SHARED((_NPAD, D), jnp.float32),
      pltpu.SemaphoreType.DMA,
      pltpu.SemaphoreType.DMA,
      pltpu.SemaphoreType.DMA,
      pltpu.SemaphoreType.DMA,
  ]

  def body(src_hbm, dst_hbm, y_hbm, zero_hbm, p0_hbm, p1_hbm,
           srcv0, srcv1, dstv0, dstv1, buf0, buf1, acc, gsem, isem,
           ssem0, ssem1):
    c = lax.axis_index("c")
    s = lax.axis_index("s")
    wid = s * 2 + c
    nchunk = _ZR // _B
    srcvs = (srcv0, srcv1)
    dstvs = (dstv0, dstv1)
    bufs = (buf0, buf1)
    nbat, base = _tile_batches(wid)

    pltpu.sync_copy(zero_hbm, buf0)

    def zinit(k, carry):
      pltpu.sync_copy(buf0, acc.at[pl.ds(s * _ZR + k * _B, _B)])
      return carry

    lax.fori_loop(0, nchunk, zinit, 0)
    plsc.subcore_barrier()

    def idx_start(j, b):
      off = (base + j) * _B
      pltpu.make_async_copy(src_hbm.at[pl.ds(off, _B)], srcvs[b], isem).start()
      pltpu.make_async_copy(dst_hbm.at[pl.ds(off, _B)], dstvs[b], isem).start()

    def idx_wait(b):
      pltpu.make_async_copy(src_hbm.at[pl.ds(0, _B)], srcvs[b], isem).wait()
      pltpu.make_async_copy(dst_hbm.at[pl.ds(0, _B)], dstvs[b], isem).wait()

    def gather_start(b):
      pltpu.make_async_copy(y_hbm.at[srcvs[b]], bufs[b], gsem).start()

    def gather_wait(b):
      pltpu.make_async_copy(y_hbm.at[srcvs[b]], bufs[b], gsem).wait()

    ssems = (ssem0, ssem1)

    def scat_start(b):
      pltpu.make_async_copy(bufs[b], acc.at[dstvs[b]], ssems[b]).start(add=True)

    def scat_wait(b):
      pltpu.make_async_copy(bufs[b], acc.at[dstvs[b]], ssems[b]).wait()

    idx_start(0, 0)
    idx_wait(0)
    gather_start(0)

    def group(g, carry):
      for b in range(2):
        j = g * 2 + b
        nxt = 1 - b

        @pl.when(j < nbat - 1)
        def _():
          idx_start(j + 1, nxt)

        gather_wait(b)

        @pl.when(j < nbat - 1)
        def _():
          idx_wait(nxt)

          @pl.when(j > 0)
          def _():
            scat_wait(nxt)

          gather_start(nxt)

        scat_start(b)
      return carry

    lax.fori_loop(0, nbat // 2, group, 0)

    @pl.when(nbat % 2 == 1)
    def _():
      scat_wait(1)
      gather_wait(0)
      scat_start(0)
      scat_wait(0)

    @pl.when(nbat % 2 == 0)
    def _():
      scat_wait(0)
      scat_wait(1)

    plsc.subcore_barrier()

    def out_chunk(k, carry):
      sl = pl.ds(s * _ZR + k * _B, _B)
      pltpu.sync_copy(acc.at[sl], buf0)

      @pl.when(c == 0)
      def _():
        pltpu.sync_copy(buf0, p0_hbm.at[sl])

      @pl.when(c == 1)
      def _():
        pltpu.sync_copy(buf0, p1_hbm.at[sl])

      return carry

    lax.fori_loop(0, nchunk, out_chunk, 0)

  return pl.kernel(body, mesh=mesh, out_type=out_type, scratch_types=scratch)


@functools.cache
def _seg_sum(D):
  return _make_seg_sum(D)


def _make_deg():
  mesh = plsc.VectorSubcoreMesh(core_axis_name="c", subcore_axis_name="s")
  out_type = [jax.ShapeDtypeStruct((_NPAD, _DW), jnp.float32),
              jax.ShapeDtypeStruct((_NPAD, _DW), jnp.float32)]
  scratch = [
      pltpu.VMEM((_B,), jnp.int32),
      pltpu.VMEM((_B,), jnp.int32),
      pltpu.VMEM((_B, _DW), jnp.float32),
      pltpu.VMEM((_B, _DW), jnp.float32),
      pltpu.VMEM_SHARED((_NPAD, _DW), jnp.float32),
      pltpu.SemaphoreType.DMA,
  ]

  def body(dst_hbm, ones_hbm, zero_hbm, d0_hbm, d1_hbm,
           dstv0, dstv1, onesv, bncv, dacc, isem):
    c = lax.axis_index("c")
    s = lax.axis_index("s")
    wid = s * 2 + c
    nchunk = _ZR // _B
    dstvs = (dstv0, dstv1)
    nbat, base = _tile_batches(wid)

    pltpu.sync_copy(zero_hbm, bncv)

    def zinit(k, carry):
      pltpu.sync_copy(bncv, dacc.at[pl.ds(s * _ZR + k * _B, _B)])
      return carry

    lax.fori_loop(0, nchunk, zinit, 0)
    pltpu.sync_copy(ones_hbm, onesv)
    plsc.subcore_barrier()

    def idx_start(j, b):
      pltpu.make_async_copy(dst_hbm.at[pl.ds((base + j) * _B, _B)],
                            dstvs[b], isem).start()

    def idx_wait(b):
      pltpu.make_async_copy(dst_hbm.at[pl.ds(0, _B)], dstvs[b], isem).wait()

    idx_start(0, 0)

    def group(g, carry):
      for b in range(2):
        j = g * 2 + b

        @pl.when(j < nbat - 1)
        def _():
          idx_start(j + 1, 1 - b)

        idx_wait(b)
        pltpu.sync_copy(onesv, dacc.at[dstvs[b]], add=True)
      return carry

    lax.fori_loop(0, nbat // 2, group, 0)

    @pl.when(nbat % 2 == 1)
    def _():
      idx_wait(0)
      pltpu.sync_copy(onesv, dacc.at[dstvs[0]], add=True)

    plsc.subcore_barrier()

    def out_chunk(k, carry):
      sl = pl.ds(s * _ZR + k * _B, _B)
      pltpu.sync_copy(dacc.at[sl], bncv)

      @pl.when(c == 0)
      def _():
        pltpu.sync_copy(bncv, d0_hbm.at[sl])

      @pl.when(c == 1)
      def _():
        pltpu.sync_copy(bncv, d1_hbm.at[sl])

      return carry

    lax.fori_loop(0, nchunk, out_chunk, 0)

  return pl.kernel(body, mesh=mesh, out_type=out_type, scratch_types=scratch)


@functools.cache
def _deg_kernel():
  return _make_deg()


def _dot_t(a, w):
  return lax.dot_general(a, w, (((1,), (1,)), ((), ())),
                         preferred_element_type=jnp.float32)


def _tc1_body(x_ref, wr_ref, b_ref, z_ref):
  z_ref[...] = _dot_t(x_ref[...], wr_ref[...]) + b_ref[0:1, :]


def _tc2_body(p0, p1, d0, d1, z1, w1l, w2r, b2, h_ref, z2_ref):
  deg = jnp.maximum((d0[...] + d1[...])[:, 0:1], 1.0)
  agg = (p0[...] + p1[...]) / deg
  h = jnp.maximum(_dot_t(agg, w1l[...]) + z1[...], 0.0)
  h_ref[...] = h
  z2_ref[...] = _dot_t(h, w2r[...]) + b2[0:1, :]


def _tc3_body(q0, q1, d0, d1, z2, w2l, out):
  deg = jnp.maximum((d0[...] + d1[...])[:, 0:1], 1.0)
  agg = (q0[...] + q1[...]) / deg
  o = _dot_t(agg, w2l[...]) + z2[...]
  e = o - jnp.max(o, axis=1, keepdims=True)
  out[...] = e - jnp.log(jnp.sum(jnp.exp(e), axis=1, keepdims=True))


def _rows(w):
  return pl.BlockSpec((_RB, w), lambda i: (i, 0))


def _full(h, w):
  return pl.BlockSpec((h, w), lambda i: (0, 0))


_tc1 = pl.pallas_call(
    _tc1_body,
    grid=(_G,),
    in_specs=[_rows(128), _full(128, 128), _full(8, 128)],
    out_specs=_rows(128),
    out_shape=jax.ShapeDtypeStruct((_N, 128), jnp.float32),
)

_tc2 = pl.pallas_call(
    _tc2_body,
    grid=(_G,),
    in_specs=[_rows(128), _rows(128), _rows(_DW), _rows(_DW), _rows(128),
              _full(128, 128), _full(64, 128), _full(8, 64)],
    out_specs=[_rows(128), _rows(64)],
    out_shape=[jax.ShapeDtypeStruct((_N, 128), jnp.float32),
               jax.ShapeDtypeStruct((_N, 64), jnp.float32)],
)

_tc3 = pl.pallas_call(
    _tc3_body,
    grid=(_G,),
    in_specs=[_rows(128), _rows(128), _rows(_DW), _rows(_DW), _rows(64),
              _full(64, 128)],
    out_specs=_rows(64),
    out_shape=jax.ShapeDtypeStruct((_N, 64), jnp.float32),
)


def kernel(x, edge_index, W1l, W1r, b1, W2l, W2r, b2):
  srcp = edge_index[0].astype(jnp.int32)
  dstp = edge_index[1].astype(jnp.int32)
  z128 = jnp.zeros((_B, 128), jnp.float32)
  zdw = jnp.zeros((_B, _DW), jnp.float32)
  onesdw = jnp.ones((_B, _DW), jnp.float32)
  b1b = jnp.broadcast_to(b1.reshape(1, -1), (8, 128))
  b2b = jnp.broadcast_to(b2.reshape(1, -1), (8, 64))

  z1 = _tc1(x, W1r, b1b)
  d0, d1 = _deg_kernel()(dstp, onesdw, zdw)
  p0, p1 = _seg_sum(128)(srcp, dstp, x, z128)
  h, z2 = _tc2(p0, p1, d0, d1, z1, W1l, W2r, b2b)
  q0, q1 = _seg_sum(128)(srcp, dstp, h, z128)
  return _tc3(q0, q1, d0, d1, z2, W2l)


# speedup vs baseline: 1.0256x; 1.0256x over previous
import functools

import jax
import jax.numpy as jnp
from jax import lax
from jax.experimental import pallas as pl
from jax.experimental.pallas import tpu as pltpu
from jax.experimental.pallas import tpu_sc as plsc

_N = 10000
_E = 320000
_B = 128
_NBAT = _E // _B
_NT = 32
_BASE_BAT = _NBAT // _NT
_EXTRA = _NBAT - _BASE_BAT * _NT
_NPAD = 10240
_ZR = _NPAD // 16
_DW = 128
_RB = 1000
_G = _N // _RB


def _tile_batches(wid):
  nbat = _BASE_BAT + (wid < _EXTRA).astype(jnp.int32)
  base = wid * _BASE_BAT + jnp.minimum(wid, _EXTRA)
  return nbat, base


def _make_seg_sum(D):
  mesh = plsc.VectorSubcoreMesh(core_axis_name="c", subcore_axis_name="s")
  out_type = [jax.ShapeDtypeStruct((_NPAD, D), jnp.float32),
              jax.ShapeDtypeStruct((_NPAD, D), jnp.float32)]
  scratch = [
      pltpu.VMEM((_B,), jnp.int32),
      pltpu.VMEM((_B,), jnp.int32),
      pltpu.VMEM((_B,), jnp.int32),
      pltpu.VMEM((_B,), jnp.int32),
      pltpu.VMEM((_B, D), jnp.float32),
      pltpu.VMEM((_B, D), jnp.float32),
      pltpu.VMEM_SHARED((_NPAD, D), jnp.float32),
      pltpu.SemaphoreType.DMA,
      pltpu.SemaphoreType.DMA,
      pltpu.SemaphoreType.DMA,
      pltpu.SemaphoreType.DMA,
  ]

  def body(edges_hbm, y_hbm, zero_hbm, p0_hbm, p1_hbm,
           srcv0, srcv1, dstv0, dstv1, buf0, buf1, acc, gsem, isem,
           ssem0, ssem1):
    c = lax.axis_index("c")
    s = lax.axis_index("s")
    wid = s * 2 + c
    nchunk = _ZR // _B
    srcvs = (srcv0, srcv1)
    dstvs = (dstv0, dstv1)
    bufs = (buf0, buf1)
    nbat, base = _tile_batches(wid)

    pltpu.sync_copy(zero_hbm, buf0)

    def zinit(k, carry):
      pltpu.sync_copy(buf0, acc.at[pl.ds(s * _ZR + k * _B, _B)])
      return carry

    lax.fori_loop(0, nchunk, zinit, 0)
    plsc.subcore_barrier()

    def idx_start(j, b):
      off = (base + j) * _B
      pltpu.make_async_copy(edges_hbm.at[pl.ds(off, _B)], srcvs[b],
                            isem).start()
      pltpu.make_async_copy(edges_hbm.at[pl.ds(_E + off, _B)], dstvs[b],
                            isem).start()

    def idx_wait(b):
      pltpu.make_async_copy(edges_hbm.at[pl.ds(0, _B)], srcvs[b], isem).wait()
      pltpu.make_async_copy(edges_hbm.at[pl.ds(0, _B)], dstvs[b], isem).wait()

    def gather_start(b):
      pltpu.make_async_copy(y_hbm.at[srcvs[b]], bufs[b], gsem).start()

    def gather_wait(b):
      pltpu.make_async_copy(y_hbm.at[srcvs[b]], bufs[b], gsem).wait()

    ssems = (ssem0, ssem1)

    def scat_start(b):
      pltpu.make_async_copy(bufs[b], acc.at[dstvs[b]], ssems[b]).start(add=True)

    def scat_wait(b):
      pltpu.make_async_copy(bufs[b], acc.at[dstvs[b]], ssems[b]).wait()

    idx_start(0, 0)
    idx_wait(0)
    gather_start(0)

    def group(g, carry):
      for b in range(2):
        j = g * 2 + b
        nxt = 1 - b

        @pl.when(j < nbat - 1)
        def _():
          idx_start(j + 1, nxt)

        gather_wait(b)

        @pl.when(j < nbat - 1)
        def _():
          idx_wait(nxt)

          @pl.when(j > 0)
          def _():
            scat_wait(nxt)

          gather_start(nxt)

        scat_start(b)
      return carry

    lax.fori_loop(0, nbat // 2, group, 0)

    @pl.when(nbat % 2 == 1)
    def _():
      scat_wait(1)
      gather_wait(0)
      scat_start(0)
      scat_wait(0)

    @pl.when(nbat % 2 == 0)
    def _():
      scat_wait(0)
      scat_wait(1)

    plsc.subcore_barrier()

    def out_chunk(k, carry):
      sl = pl.ds(s * _ZR + k * _B, _B)
      pltpu.sync_copy(acc.at[sl], buf0)

      @pl.when(c == 0)
      def _():
        pltpu.sync_copy(buf0, p0_hbm.at[sl])

      @pl.when(c == 1)
      def _():
        pltpu.sync_copy(buf0, p1_hbm.at[sl])

      return carry

    lax.fori_loop(0, nchunk, out_chunk, 0)

  return pl.kernel(body, mesh=mesh, out_type=out_type, scratch_types=scratch)


@functools.cache
def _seg_sum(D):
  return _make_seg_sum(D)


def _make_deg():
  mesh = plsc.VectorSubcoreMesh(core_axis_name="c", subcore_axis_name="s")
  out_type = [jax.ShapeDtypeStruct((_NPAD, _DW), jnp.float32),
              jax.ShapeDtypeStruct((_NPAD, _DW), jnp.float32)]
  scratch = [
      pltpu.VMEM((_B,), jnp.int32),
      pltpu.VMEM((_B,), jnp.int32),
      pltpu.VMEM((_B, _DW), jnp.float32),
      pltpu.VMEM((_B, _DW), jnp.float32),
      pltpu.VMEM_SHARED((_NPAD, _DW), jnp.float32),
      pltpu.SemaphoreType.DMA,
  ]

  def body(edges_hbm, ones_hbm, zero_hbm, d0_hbm, d1_hbm,
           dstv0, dstv1, onesv, bncv, dacc, isem):
    c = lax.axis_index("c")
    s = lax.axis_index("s")
    wid = s * 2 + c
    nchunk = _ZR // _B
    dstvs = (dstv0, dstv1)
    nbat, base = _tile_batches(wid)

    pltpu.sync_copy(zero_hbm, bncv)

    def zinit(k, carry):
      pltpu.sync_copy(bncv, dacc.at[pl.ds(s * _ZR + k * _B, _B)])
      return carry

    lax.fori_loop(0, nchunk, zinit, 0)
    pltpu.sync_copy(ones_hbm, onesv)
    plsc.subcore_barrier()

    def idx_start(j, b):
      pltpu.make_async_copy(edges_hbm.at[pl.ds(_E + (base + j) * _B, _B)],
                            dstvs[b], isem).start()

    def idx_wait(b):
      pltpu.make_async_copy(edges_hbm.at[pl.ds(0, _B)], dstvs[b], isem).wait()

    idx_start(0, 0)

    def group(g, carry):
      for b in range(2):
        j = g * 2 + b

        @pl.when(j < nbat - 1)
        def _():
          idx_start(j + 1, 1 - b)

        idx_wait(b)
        pltpu.sync_copy(onesv, dacc.at[dstvs[b]], add=True)
      return carry

    lax.fori_loop(0, nbat // 2, group, 0)

    @pl.when(nbat % 2 == 1)
    def _():
      idx_wait(0)
      pltpu.sync_copy(onesv, dacc.at[dstvs[0]], add=True)

    plsc.subcore_barrier()

    def out_chunk(k, carry):
      sl = pl.ds(s * _ZR + k * _B, _B)
      pltpu.sync_copy(dacc.at[sl], bncv)

      @pl.when(c == 0)
      def _():
        pltpu.sync_copy(bncv, d0_hbm.at[sl])

      @pl.when(c == 1)
      def _():
        pltpu.sync_copy(bncv, d1_hbm.at[sl])

      return carry

    lax.fori_loop(0, nchunk, out_chunk, 0)

  return pl.kernel(body, mesh=mesh, out_type=out_type, scratch_types=scratch)


@functools.cache
def _deg_kernel():
  return _make_deg()


def _dot_t(a, w):
  return lax.dot_general(a, w, (((1,), (1,)), ((), ())),
                         preferred_element_type=jnp.float32)


def _tc1_body(x_ref, wr_ref, b_ref, z_ref):
  z_ref[...] = _dot_t(x_ref[...], wr_ref[...]) + b_ref[0:1, :]


def _tc2_body(p0, p1, d0, d1, z1, w1l, w2r, b2, h_ref, z2_ref):
  deg = jnp.maximum((d0[...] + d1[...])[:, 0:1], 1.0)
  agg = (p0[...] + p1[...]) / deg
  h = jnp.maximum(_dot_t(agg, w1l[...]) + z1[...], 0.0)
  h_ref[...] = h
  z2_ref[...] = _dot_t(h, w2r[...]) + b2[0:1, :]


def _tc3_body(q0, q1, d0, d1, z2, w2l, out):
  deg = jnp.maximum((d0[...] + d1[...])[:, 0:1], 1.0)
  agg = (q0[...] + q1[...]) / deg
  o = _dot_t(agg, w2l[...]) + z2[...]
  e = o - jnp.max(o, axis=1, keepdims=True)
  out[...] = e - jnp.log(jnp.sum(jnp.exp(e), axis=1, keepdims=True))


def _rows(w):
  return pl.BlockSpec((_RB, w), lambda i: (i, 0))


def _full(h, w):
  return pl.BlockSpec((h, w), lambda i: (0, 0))


_tc1 = pl.pallas_call(
    _tc1_body,
    grid=(_G,),
    in_specs=[_rows(128), _full(128, 128), _full(8, 128)],
    out_specs=_rows(128),
    out_shape=jax.ShapeDtypeStruct((_N, 128), jnp.float32),
)

_tc2 = pl.pallas_call(
    _tc2_body,
    grid=(_G,),
    in_specs=[_rows(128), _rows(128), _rows(_DW), _rows(_DW), _rows(128),
              _full(128, 128), _full(64, 128), _full(8, 64)],
    out_specs=[_rows(128), _rows(64)],
    out_shape=[jax.ShapeDtypeStruct((_N, 128), jnp.float32),
               jax.ShapeDtypeStruct((_N, 64), jnp.float32)],
)

_tc3 = pl.pallas_call(
    _tc3_body,
    grid=(_G,),
    in_specs=[_rows(128), _rows(128), _rows(_DW), _rows(_DW), _rows(64),
              _full(64, 128)],
    out_specs=_rows(64),
    out_shape=jax.ShapeDtypeStruct((_N, 64), jnp.float32),
)


def kernel(x, edge_index, W1l, W1r, b1, W2l, W2r, b2):
  edges = edge_index.astype(jnp.int32).reshape(2 * _E)
  z128 = jnp.zeros((_B, 128), jnp.float32)
  zdw = jnp.zeros((_B, _DW), jnp.float32)
  onesdw = jnp.ones((_B, _DW), jnp.float32)
  b1b = jnp.broadcast_to(b1.reshape(1, -1), (8, 128))
  b2b = jnp.broadcast_to(b2.reshape(1, -1), (8, 64))

  z1 = _tc1(x, W1r, b1b)
  d0, d1 = _deg_kernel()(edges, onesdw, zdw)
  p0, p1 = _seg_sum(128)(edges, x, z128)
  h, z2 = _tc2(p0, p1, d0, d1, z1, W1l, W2r, b2b)
  q0, q1 = _seg_sum(128)(edges, h, z128)
  return _tc3(q0, q1, d0, d1, z2, W2l)
